# baseline (speedup 1.0000x reference)
import functools

import jax
import jax.numpy as jnp
from jax.experimental import pallas as pl
from jax.experimental.pallas import tpu as pltpu
from jax.experimental.pallas import tpu_sc as plsc

DIM = 2048
NHEAD = 16
QK = 128
VD = 128
NEXP = 16
TOPK = 2
INTER = 1408
SH_INTER = 2816
EPS = 1e-8
S = 2048
F32 = jnp.float32

BM = 256
BQ = 512
TR = 256
NT = (TOPK * S + NEXP * (TR - 1) + TR - 1) // TR
PMAX = NT * TR
FC = INTER // 2
NSH = 11
BSH = SH_INTER // NSH
NZ = 2
BZ = DIM // NZ


def _dt(a, b):
    return jax.lax.dot_general(a, b, (((1,), (1,)), ((), ())),
                               preferred_element_type=F32)


def _dtb(a, b):
    return jax.lax.dot_general(a.astype(jnp.bfloat16), b.astype(jnp.bfloat16),
                               (((1,), (1,)), ((), ())),
                               preferred_element_type=F32)


def _ln_body(x_ref, w_ref, o_ref):
    x = x_ref[...]
    mu = jnp.mean(x, axis=-1, keepdims=True)
    var = jnp.mean((x - mu) ** 2, axis=-1, keepdims=True)
    o_ref[...] = (x - mu) * jax.lax.rsqrt(var + EPS) * w_ref[...]


def _ln(x, w):
    return pl.pallas_call(
        _ln_body,
        grid=(S // BM,),
        in_specs=[pl.BlockSpec((BM, DIM), lambda i: (i, 0)),
                  pl.BlockSpec((1, DIM), lambda i: (0, 0))],
        out_specs=pl.BlockSpec((BM, DIM), lambda i: (i, 0)),
        out_shape=jax.ShapeDtypeStruct((S, DIM), F32),
    )(x, w.reshape(1, DIM))


def _mm_body(x_ref, w_ref, b_ref, o_ref):
    o_ref[...] = _dt(x_ref[...], w_ref[...]) + b_ref[...]


def _lnmm_body(x_ref, lw_ref, w_ref, b_ref, h_ref, o_ref):
    x = x_ref[...]
    mu = jnp.mean(x, axis=-1, keepdims=True)
    var = jnp.mean((x - mu) ** 2, axis=-1, keepdims=True)
    h = (x - mu) * jax.lax.rsqrt(var + EPS) * lw_ref[...]
    h_ref[...] = h
    o_ref[...] = _dt(h, w_ref[...]) + b_ref[...]


def _lnmm(x, lw, w, b):
    n = w.shape[0]
    return pl.pallas_call(
        _lnmm_body,
        grid=(S // BM,),
        in_specs=[pl.BlockSpec((BM, DIM), lambda i: (i, 0)),
                  pl.BlockSpec((1, DIM), lambda i: (0, 0)),
                  pl.BlockSpec((n, DIM), lambda i: (0, 0)),
                  pl.BlockSpec((1, n), lambda i: (0, 0))],
        out_specs=(pl.BlockSpec((BM, DIM), lambda i: (i, 0)),
                   pl.BlockSpec((BM, n), lambda i: (i, 0))),
        out_shape=(jax.ShapeDtypeStruct((S, DIM), F32),
                   jax.ShapeDtypeStruct((S, n), F32)),
    )(x, lw.reshape(1, DIM), w, b.reshape(1, n))


def _mm(x, w, b):
    n = w.shape[0]
    return pl.pallas_call(
        _mm_body,
        grid=(S // BM,),
        in_specs=[pl.BlockSpec((BM, DIM), lambda i: (i, 0)),
                  pl.BlockSpec((n, DIM), lambda i: (0, 0)),
                  pl.BlockSpec((1, n), lambda i: (0, 0))],
        out_specs=pl.BlockSpec((BM, n), lambda i: (i, 0)),
        out_shape=jax.ShapeDtypeStruct((S, n), F32),
    )(x, w, b.reshape(1, n))


def _mm_res_body(x_ref, w_ref, b_ref, r_ref, lw_ref, o_ref, h_ref):
    xa = _dt(x_ref[...], w_ref[...]) + b_ref[...] + r_ref[...]
    o_ref[...] = xa
    mu = jnp.mean(xa, axis=-1, keepdims=True)
    var = jnp.mean((xa - mu) ** 2, axis=-1, keepdims=True)
    h_ref[...] = (xa - mu) * jax.lax.rsqrt(var + EPS) * lw_ref[...]


def _mm_res(x, w, b, r, lw):
    n = w.shape[0]
    return pl.pallas_call(
        _mm_res_body,
        grid=(S // BM,),
        in_specs=[pl.BlockSpec((BM, DIM), lambda i: (i, 0)),
                  pl.BlockSpec((n, DIM), lambda i: (0, 0)),
                  pl.BlockSpec((1, n), lambda i: (0, 0)),
                  pl.BlockSpec((BM, n), lambda i: (i, 0)),
                  pl.BlockSpec((1, DIM), lambda i: (0, 0))],
        out_specs=(pl.BlockSpec((BM, n), lambda i: (i, 0)),
                   pl.BlockSpec((BM, DIM), lambda i: (i, 0))),
        out_shape=(jax.ShapeDtypeStruct((S, n), F32),
                   jax.ShapeDtypeStruct((S, DIM), F32)),
    )(x, w, b.reshape(1, n), r, lw.reshape(1, DIM))


def _attn_body(q_ref, k_ref, v_ref, m_ref, o_ref):
    h = pl.program_id(0)
    qi = pl.program_id(1)
    s = _dt(q_ref[...], k_ref[...]) * (QK ** -0.5)
    i = qi * BQ + jax.lax.broadcasted_iota(jnp.int32, (BQ, S), 0)
    j = jax.lax.broadcasted_iota(jnp.int32, (BQ, S), 1)
    hf = jax.lax.convert_element_type(h + 1, F32)
    bias = jnp.where(i >= j,
                     jax.lax.convert_element_type(j - i, F32) * jnp.exp2(-hf),
                     0.0)
    s = s + bias + m_ref[pl.ds(qi * BQ, BQ), :]
    s = s - jnp.max(s, axis=-1, keepdims=True)
    e = jnp.exp(s)
    p = e / jnp.sum(e, axis=-1, keepdims=True)
    pv = jax.lax.dot_general(p, v_ref[...], (((0,), (0,)), ((), ())),
                             preferred_element_type=F32)

    @pl.when(qi == 0)
    def _init():
        o_ref[...] = pv

    @pl.when(qi > 0)
    def _acc():
        o_ref[...] += pv


def _attn(q, k, v, mask):
    return pl.pallas_call(
        _attn_body,
        grid=(NHEAD, S // BQ),
        in_specs=[pl.BlockSpec((BQ, QK), lambda h, i: (i, h)),
                  pl.BlockSpec((S, QK), lambda h, i: (0, h)),
                  pl.BlockSpec((BQ, VD), lambda h, i: (i, h)),
                  pl.BlockSpec((S, S), lambda h, i: (0, 0))],
        out_specs=pl.BlockSpec((S, VD), lambda h, i: (0, h)),
        out_shape=jax.ShapeDtypeStruct((S, NHEAD * VD), F32),
    )(q, k, v, mask)


def _gate_body(x_ref, w_ref, b_ref, o_ref, t_ref):
    logits = jax.lax.dot_general(x_ref[...], w_ref[...],
                                 (((1,), (1,)), ((), ())),
                                 preferred_element_type=F32,
                                 precision=jax.lax.Precision.HIGHEST)
    lane = jax.lax.broadcasted_iota(jnp.int32, (S, 128), 1)
    neg = jnp.float32(-1e30)
    logits = jnp.where(lane < NEXP, logits, neg)
    m = jnp.max(logits, axis=-1, keepdims=True)
    e = jnp.exp(logits - m)
    p = e / jnp.sum(e, axis=-1, keepdims=True)
    biased = jnp.where(lane < NEXP, p + b_ref[...], neg)
    m1 = jnp.max(biased, axis=-1, keepdims=True)
    i1 = jnp.min(jnp.where(biased == m1, lane, 127), axis=-1, keepdims=True)
    b2 = jnp.where(lane == i1, neg, biased)
    m2 = jnp.max(b2, axis=-1, keepdims=True)
    i2 = jnp.min(jnp.where(b2 == m2, lane, 127), axis=-1, keepdims=True)
    w1 = jnp.sum(jnp.where(lane == i1, p, 0.0), axis=-1, keepdims=True)
    w2 = jnp.sum(jnp.where(lane == i2, p, 0.0), axis=-1, keepdims=True)

    oh0 = jnp.where(lane == i1, 1.0, 0.0)
    oh1 = jnp.where(lane == i2, 1.0, 0.0)
    ti = jax.lax.broadcasted_iota(jnp.int32, (S, S), 0)
    tj = jax.lax.broadcasted_iota(jnp.int32, (S, S), 1)
    ltri = jnp.where(ti >= tj, 1.0, 0.0).astype(jnp.bfloat16)
    c0 = jax.lax.dot_general(ltri, oh0.astype(jnp.bfloat16),
                             (((1,), (0,)), ((), ())),
                             preferred_element_type=F32)
    c1 = jax.lax.dot_general(ltri, oh1.astype(jnp.bfloat16),
                             (((1,), (0,)), ((), ())),
                             preferred_element_type=F32)
    c0tot = c0[S - 1:S, :]
    c1tot = c1[S - 1:S, :]
    counts = c0tot + c1tot
    pcounts = jnp.floor((counts + (TR - 1)) * (1.0 / TR)) * TR
    li = jax.lax.broadcasted_iota(jnp.int32, (128, 128), 0)
    lj = jax.lax.broadcasted_iota(jnp.int32, (128, 128), 1)
    sutri = jnp.where(li < lj, 1.0, 0.0).astype(jnp.bfloat16)
    pstarts = jax.lax.dot_general(pcounts.astype(jnp.bfloat16), sutri,
                                  (((1,), (0,)), ((), ())),
                                  preferred_element_type=F32)
    rank0 = c0 - oh0
    rank1 = c0tot + c1 - oh1
    ppos0 = jnp.sum(oh0 * (pstarts + rank0), axis=-1, keepdims=True)
    ppos1 = jnp.sum(oh1 * (pstarts + rank1), axis=-1, keepdims=True)
    o_ref[...] = (jnp.where(lane == 0, w1, 0.0)
                  + jnp.where(lane == 1, w2, 0.0)
                  + jnp.where(lane == 2, ppos0, 0.0)
                  + jnp.where(lane == 3, ppos1, 0.0))
    ni = jax.lax.broadcasted_iota(jnp.int32, (NT, 128), 0)
    nl = jax.lax.broadcasted_iota(jnp.int32, (NT, 128), 1)
    tstart = jax.lax.convert_element_type(ni * TR, F32)
    ple = jnp.where(nl < NEXP, jnp.where(pstarts <= tstart, 1.0, 0.0), 0.0)
    te = jnp.sum(ple, axis=-1, keepdims=True) - 1.0
    t_ref[...] = jnp.where(nl == 0, te, 0.0)


def _gate(x, gw_pad, gb_pad):
    return pl.pallas_call(
        _gate_body,
        grid=(1,),
        in_specs=[pl.BlockSpec((S, DIM), lambda i: (0, 0)),
                  pl.BlockSpec((128, DIM), lambda i: (0, 0)),
                  pl.BlockSpec((1, 128), lambda i: (0, 0))],
        out_specs=(pl.BlockSpec((S, 128), lambda i: (0, 0)),
                   pl.BlockSpec((NT, 128), lambda i: (0, 0))),
        out_shape=(jax.ShapeDtypeStruct((S, 128), F32),
                   jax.ShapeDtypeStruct((NT, 128), F32)),
    )(x, gw_pad, gb_pad)


def _f1_body(te_ref, xg_ref, w1_ref, w3_ref, b1_ref, b3_ref, o_ref):
    x = xg_ref[...]
    h1 = _dtb(x, w1_ref[0, 0]) + b1_ref[0]
    h3 = _dtb(x, w3_ref[0, 0]) + b3_ref[0]
    o_ref[...] = h1 * jax.nn.sigmoid(h1) * h3


def _moe_f1_half(te, xg, ew1, ew3, eb1, eb3, c):
    grid_spec = pltpu.PrefetchScalarGridSpec(
        num_scalar_prefetch=1,
        grid=(NT,),
        in_specs=[
            pl.BlockSpec((TR, DIM), lambda t, te: (t, 0)),
            pl.BlockSpec((1, 1, FC, DIM), lambda t, te: (te[t], c, 0, 0)),
            pl.BlockSpec((1, 1, FC, DIM), lambda t, te: (te[t], c, 0, 0)),
            pl.BlockSpec((1, 1, FC), lambda t, te: (te[t], 0, 0)),
            pl.BlockSpec((1, 1, FC), lambda t, te: (te[t], 0, 0)),
        ],
        out_specs=pl.BlockSpec((TR, FC), lambda t, te: (t, 0)),
    )
    return pl.pallas_call(
        _f1_body, grid_spec=grid_spec,
        out_shape=jax.ShapeDtypeStruct((PMAX, FC), F32),
    )(te, xg,
      ew1.reshape(NEXP, 2, FC, DIM), ew3.reshape(NEXP, 2, FC, DIM),
      eb1[:, c * FC:(c + 1) * FC].reshape(NEXP, 1, FC),
      eb3[:, c * FC:(c + 1) * FC].reshape(NEXP, 1, FC))


def _f2_body(te_ref, a0_ref, a1_ref, w_ref, b_ref, o_ref):
    a = jnp.concatenate([a0_ref[...], a1_ref[...]], axis=-1)
    o_ref[...] = _dtb(a, w_ref[0]) + b_ref[0]


def _moe_f2(te, act0, act1, ew2, eb2):
    grid_spec = pltpu.PrefetchScalarGridSpec(
        num_scalar_prefetch=1,
        grid=(NT,),
        in_specs=[
            pl.BlockSpec((TR, FC), lambda t, te: (t, 0)),
            pl.BlockSpec((TR, FC), lambda t, te: (t, 0)),
            pl.BlockSpec((1, DIM, INTER), lambda t, te: (te[t], 0, 0)),
            pl.BlockSpec((1, 1, DIM), lambda t, te: (te[t], 0, 0)),
        ],
        out_specs=pl.BlockSpec((TR, DIM), lambda t, te: (t, 0)),
    )
    return pl.pallas_call(
        _f2_body, grid_spec=grid_spec,
        out_shape=jax.ShapeDtypeStruct((PMAX, DIM), F32),
    )(te, act0, act1, ew2, eb2.reshape(NEXP, 1, DIM))


_SC_CORES = 2
_SC_SUBCORES = 16
_SC_NW = _SC_CORES * _SC_SUBCORES


def _sc_gather(table, idx, nrows, chunk):
    nper = nrows // _SC_NW
    mesh = plsc.VectorSubcoreMesh(core_axis_name="c", subcore_axis_name="s")

    @functools.partial(
        pl.kernel,
        out_type=jax.ShapeDtypeStruct((nrows, DIM), F32),
        mesh=mesh,
        scratch_types=[
            pltpu.VMEM((chunk,), jnp.int32),
            pltpu.VMEM((chunk, DIM), F32),
            pltpu.SemaphoreType.DMA,
        ],
    )
    def gk(table_hbm, idx_hbm, out_hbm, idx_v, rows_v, sem):
        wid = jax.lax.axis_index("s") * _SC_CORES + jax.lax.axis_index("c")
        base = wid * nper

        def body(j, carry):
            off = base + j * chunk
            pltpu.sync_copy(idx_hbm.at[pl.ds(off, chunk)], idx_v)
            pltpu.async_copy(table_hbm.at[idx_v], rows_v, sem).wait()
            pltpu.sync_copy(rows_v, out_hbm.at[pl.ds(off, chunk)])
            return carry

        jax.lax.fori_loop(0, nper // chunk, body, 0)

    return gk(table, idx)


def _dmm_body(x_ref, w1_ref, b1_ref, w3_ref, b3_ref, o_ref):
    x = x_ref[...]
    h1 = _dtb(x, w1_ref[...]) + b1_ref[...]
    h3 = _dtb(x, w3_ref[...]) + b3_ref[...]
    o_ref[...] = h1 * jax.nn.sigmoid(h1) * h3


def _sh_act(x, sw1, sb1, sw3, sb3):
    return pl.pallas_call(
        _dmm_body,
        grid=(NSH, S // BM),
        in_specs=[pl.BlockSpec((BM, DIM), lambda n, m: (m, 0)),
                  pl.BlockSpec((BSH, DIM), lambda n, m: (n, 0)),
                  pl.BlockSpec((1, BSH), lambda n, m: (0, n)),
                  pl.BlockSpec((BSH, DIM), lambda n, m: (n, 0)),
                  pl.BlockSpec((1, BSH), lambda n, m: (0, n))],
        out_specs=pl.BlockSpec((BM, BSH), lambda n, m: (m, n)),
        out_shape=jax.ShapeDtypeStruct((S, SH_INTER), F32),
    )(x, sw1, sb1.reshape(1, SH_INTER), sw3, sb3.reshape(1, SH_INTER))


def _zf_body(a_ref, w_ref, b_ref, r_ref, g_ref, y0_ref, y1_ref, o_ref):
    o_ref[...] = (_dtb(a_ref[...], w_ref[...]) + b_ref[...] + r_ref[...]
                  + g_ref[:, 0:1] * y0_ref[...] + g_ref[:, 1:2] * y1_ref[...])


def _z_final(a, sw2, sb2, xa, gt, yg0, yg1):
    return pl.pallas_call(
        _zf_body,
        grid=(NZ, S // BM),
        in_specs=[pl.BlockSpec((BM, SH_INTER), lambda n, m: (m, 0)),
                  pl.BlockSpec((BZ, SH_INTER), lambda n, m: (n, 0)),
                  pl.BlockSpec((1, BZ), lambda n, m: (0, n)),
                  pl.BlockSpec((BM, BZ), lambda n, m: (m, n)),
                  pl.BlockSpec((BM, 128), lambda n, m: (m, 0)),
                  pl.BlockSpec((BM, BZ), lambda n, m: (m, n)),
                  pl.BlockSpec((BM, BZ), lambda n, m: (m, n))],
        out_specs=pl.BlockSpec((BM, BZ), lambda n, m: (m, n)),
        out_shape=jax.ShapeDtypeStruct((S, DIM), F32),
    )(a, sw2, sb2.reshape(1, DIM), xa, gt, yg0, yg1)


def kernel(x, start_pos, mask, wq_w, wq_b, wk_w, wk_b, wv_w, wv_b, wo_w, wo_b,
           an_w, fn_w, g_w, g_b, ew1, eb1, ew2, eb2, ew3, eb3,
           sw1, sb1, sw2, sb2, sw3, sb3):
    x2 = x[0]
    h, q = _lnmm(x2, an_w, wq_w, wq_b)
    k = _mm(h, wk_w, wk_b)
    v = _mm(h, wv_w, wv_b)
    ao = _attn(q, k, v, mask)
    xa, h2 = _mm_res(ao, wo_w, wo_b, x2, fn_w)

    gw_pad = jnp.zeros((128, DIM), F32).at[:NEXP].set(g_w)
    gb_pad = jnp.zeros((1, 128), F32).at[0, :NEXP].set(g_b)
    gt, te_out = _gate(h2, gw_pad, gb_pad)
    pos0 = gt[:, 2].astype(jnp.int32)
    pos1 = gt[:, 3].astype(jnp.int32)
    tile_e = te_out[:, 0].astype(jnp.int32)
    toks = jnp.arange(S, dtype=jnp.int32)
    tok_pad = (jnp.zeros((PMAX,), jnp.int32)
               .at[jnp.concatenate([pos0, pos1])]
               .set(jnp.concatenate([toks, toks])))

    xg = _sc_gather(h2, tok_pad, PMAX, 32)
    act0 = _moe_f1_half(tile_e, xg, ew1, ew3, eb1, eb3, 0)
    act1 = _moe_f1_half(tile_e, xg, ew1, ew3, eb1, eb3, 1)
    ye = _moe_f2(tile_e, act0, act1, ew2, eb2)
    yg0 = jnp.take(ye, pos0, axis=0)
    yg1 = jnp.take(ye, pos1, axis=0)

    sh = _sh_act(h2, sw1, sb1, sw3, sb3)
    out = _z_final(sh, sw2, sb2, xa, gt, yg0, yg1)
    return out[None]


# speedup vs baseline: 1.2384x; 1.2384x over previous
import jax
import jax.numpy as jnp
from jax.experimental import pallas as pl
from jax.experimental.pallas import tpu as pltpu

DIM = 2048
NHEAD = 16
QK = 128
VD = 128
NEXP = 16
TOPK = 2
INTER = 1408
SH_INTER = 2816
EPS = 1e-8
S = 2048
F32 = jnp.float32

BM = 256
BQ = 512
TR = 256
NT = (TOPK * S + NEXP * (TR - 1) + TR - 1) // TR
PMAX = NT * TR
FC = INTER // 2
NSH = 11
BSH = SH_INTER // NSH
NZ = 2
BZ = DIM // NZ


def _dt(a, b):
    return jax.lax.dot_general(a, b, (((1,), (1,)), ((), ())),
                               preferred_element_type=F32)


def _dtb(a, b):
    return jax.lax.dot_general(a.astype(jnp.bfloat16), b.astype(jnp.bfloat16),
                               (((1,), (1,)), ((), ())),
                               preferred_element_type=F32)


def _ln_body(x_ref, w_ref, o_ref):
    x = x_ref[...]
    mu = jnp.mean(x, axis=-1, keepdims=True)
    var = jnp.mean((x - mu) ** 2, axis=-1, keepdims=True)
    o_ref[...] = (x - mu) * jax.lax.rsqrt(var + EPS) * w_ref[...]


def _ln(x, w):
    return pl.pallas_call(
        _ln_body,
        grid=(S // BM,),
        in_specs=[pl.BlockSpec((BM, DIM), lambda i: (i, 0)),
                  pl.BlockSpec((1, DIM), lambda i: (0, 0))],
        out_specs=pl.BlockSpec((BM, DIM), lambda i: (i, 0)),
        out_shape=jax.ShapeDtypeStruct((S, DIM), F32),
    )(x, w.reshape(1, DIM))


def _mm_body(x_ref, w_ref, b_ref, o_ref):
    o_ref[...] = _dt(x_ref[...], w_ref[...]) + b_ref[...]


def _lnmm_body(x_ref, lw_ref, w_ref, b_ref, h_ref, o_ref):
    x = x_ref[...]
    mu = jnp.mean(x, axis=-1, keepdims=True)
    var = jnp.mean((x - mu) ** 2, axis=-1, keepdims=True)
    h = (x - mu) * jax.lax.rsqrt(var + EPS) * lw_ref[...]
    h_ref[...] = h
    o_ref[...] = _dt(h, w_ref[...]) + b_ref[...]


def _lnmm(x, lw, w, b):
    n = w.shape[0]
    return pl.pallas_call(
        _lnmm_body,
        grid=(S // BM,),
        in_specs=[pl.BlockSpec((BM, DIM), lambda i: (i, 0)),
                  pl.BlockSpec((1, DIM), lambda i: (0, 0)),
                  pl.BlockSpec((n, DIM), lambda i: (0, 0)),
                  pl.BlockSpec((1, n), lambda i: (0, 0))],
        out_specs=(pl.BlockSpec((BM, DIM), lambda i: (i, 0)),
                   pl.BlockSpec((BM, n), lambda i: (i, 0))),
        out_shape=(jax.ShapeDtypeStruct((S, DIM), F32),
                   jax.ShapeDtypeStruct((S, n), F32)),
    )(x, lw.reshape(1, DIM), w, b.reshape(1, n))


def _mm(x, w, b):
    n = w.shape[0]
    return pl.pallas_call(
        _mm_body,
        grid=(S // BM,),
        in_specs=[pl.BlockSpec((BM, DIM), lambda i: (i, 0)),
                  pl.BlockSpec((n, DIM), lambda i: (0, 0)),
                  pl.BlockSpec((1, n), lambda i: (0, 0))],
        out_specs=pl.BlockSpec((BM, n), lambda i: (i, 0)),
        out_shape=jax.ShapeDtypeStruct((S, n), F32),
    )(x, w, b.reshape(1, n))


def _mm_res_body(x_ref, w_ref, b_ref, r_ref, lw_ref, o_ref, h_ref):
    xa = _dt(x_ref[...], w_ref[...]) + b_ref[...] + r_ref[...]
    o_ref[...] = xa
    mu = jnp.mean(xa, axis=-1, keepdims=True)
    var = jnp.mean((xa - mu) ** 2, axis=-1, keepdims=True)
    h_ref[...] = (xa - mu) * jax.lax.rsqrt(var + EPS) * lw_ref[...]


def _mm_res(x, w, b, r, lw):
    n = w.shape[0]
    return pl.pallas_call(
        _mm_res_body,
        grid=(S // BM,),
        in_specs=[pl.BlockSpec((BM, DIM), lambda i: (i, 0)),
                  pl.BlockSpec((n, DIM), lambda i: (0, 0)),
                  pl.BlockSpec((1, n), lambda i: (0, 0)),
                  pl.BlockSpec((BM, n), lambda i: (i, 0)),
                  pl.BlockSpec((1, DIM), lambda i: (0, 0))],
        out_specs=(pl.BlockSpec((BM, n), lambda i: (i, 0)),
                   pl.BlockSpec((BM, DIM), lambda i: (i, 0))),
        out_shape=(jax.ShapeDtypeStruct((S, n), F32),
                   jax.ShapeDtypeStruct((S, DIM), F32)),
    )(x, w, b.reshape(1, n), r, lw.reshape(1, DIM))


def _attn_body(q_ref, k_ref, v_ref, m_ref, o_ref):
    h = pl.program_id(0)
    qi = pl.program_id(1)
    s = _dt(q_ref[...], k_ref[...]) * (QK ** -0.5)
    i = qi * BQ + jax.lax.broadcasted_iota(jnp.int32, (BQ, S), 0)
    j = jax.lax.broadcasted_iota(jnp.int32, (BQ, S), 1)
    hf = jax.lax.convert_element_type(h + 1, F32)
    bias = jnp.where(i >= j,
                     jax.lax.convert_element_type(j - i, F32) * jnp.exp2(-hf),
                     0.0)
    s = s + bias + m_ref[pl.ds(qi * BQ, BQ), :]
    s = s - jnp.max(s, axis=-1, keepdims=True)
    e = jnp.exp(s)
    p = e / jnp.sum(e, axis=-1, keepdims=True)
    pv = jax.lax.dot_general(p, v_ref[...], (((0,), (0,)), ((), ())),
                             preferred_element_type=F32)

    @pl.when(qi == 0)
    def _init():
        o_ref[...] = pv

    @pl.when(qi > 0)
    def _acc():
        o_ref[...] += pv


def _attn(q, k, v, mask):
    return pl.pallas_call(
        _attn_body,
        grid=(NHEAD, S // BQ),
        in_specs=[pl.BlockSpec((BQ, QK), lambda h, i: (i, h)),
                  pl.BlockSpec((S, QK), lambda h, i: (0, h)),
                  pl.BlockSpec((BQ, VD), lambda h, i: (i, h)),
                  pl.BlockSpec((S, S), lambda h, i: (0, 0))],
        out_specs=pl.BlockSpec((S, VD), lambda h, i: (0, h)),
        out_shape=jax.ShapeDtypeStruct((S, NHEAD * VD), F32),
    )(q, k, v, mask)


def _gate_body(x_ref, w_ref, b_ref, o_ref, t_ref):
    logits16 = jax.lax.dot_general(x_ref[...], w_ref[...],
                                   (((1,), (1,)), ((), ())),
                                   preferred_element_type=F32,
                                   precision=jax.lax.Precision.HIGHEST)
    lane = jax.lax.broadcasted_iota(jnp.int32, (S, 128), 1)
    neg = jnp.float32(-1e30)
    logits = jnp.where(lane < NEXP,
                       jnp.pad(logits16, ((0, 0), (0, 128 - NEXP))), neg)
    m = jnp.max(logits, axis=-1, keepdims=True)
    e = jnp.exp(logits - m)
    p = e / jnp.sum(e, axis=-1, keepdims=True)
    bias_row = jnp.pad(b_ref[...], ((0, 0), (0, 128 - NEXP)))
    biased = jnp.where(lane < NEXP, p + bias_row, neg)
    m1 = jnp.max(biased, axis=-1, keepdims=True)
    i1 = jnp.min(jnp.where(biased == m1, lane, 127), axis=-1, keepdims=True)
    b2 = jnp.where(lane == i1, neg, biased)
    m2 = jnp.max(b2, axis=-1, keepdims=True)
    i2 = jnp.min(jnp.where(b2 == m2, lane, 127), axis=-1, keepdims=True)
    w1 = jnp.sum(jnp.where(lane == i1, p, 0.0), axis=-1, keepdims=True)
    w2 = jnp.sum(jnp.where(lane == i2, p, 0.0), axis=-1, keepdims=True)

    oh0 = jnp.where(lane == i1, 1.0, 0.0)
    oh1 = jnp.where(lane == i2, 1.0, 0.0)
    ti = jax.lax.broadcasted_iota(jnp.int32, (S, S), 0)
    tj = jax.lax.broadcasted_iota(jnp.int32, (S, S), 1)
    ltri = jnp.where(ti >= tj, 1.0, 0.0).astype(jnp.bfloat16)
    c0 = jax.lax.dot_general(ltri, oh0.astype(jnp.bfloat16),
                             (((1,), (0,)), ((), ())),
                             preferred_element_type=F32)
    c1 = jax.lax.dot_general(ltri, oh1.astype(jnp.bfloat16),
                             (((1,), (0,)), ((), ())),
                             preferred_element_type=F32)
    c0tot = c0[S - 1:S, :]
    c1tot = c1[S - 1:S, :]
    counts = c0tot + c1tot
    pcounts = jnp.floor((counts + (TR - 1)) * (1.0 / TR)) * TR
    li = jax.lax.broadcasted_iota(jnp.int32, (128, 128), 0)
    lj = jax.lax.broadcasted_iota(jnp.int32, (128, 128), 1)
    sutri = jnp.where(li < lj, 1.0, 0.0).astype(jnp.bfloat16)
    pstarts = jax.lax.dot_general(pcounts.astype(jnp.bfloat16), sutri,
                                  (((1,), (0,)), ((), ())),
                                  preferred_element_type=F32)
    rank0 = c0 - oh0
    rank1 = c0tot + c1 - oh1
    ppos0 = jnp.sum(oh0 * (pstarts + rank0), axis=-1, keepdims=True)
    ppos1 = jnp.sum(oh1 * (pstarts + rank1), axis=-1, keepdims=True)
    o_ref[...] = (jnp.where(lane == 0, w1, 0.0)
                  + jnp.where(lane == 1, w2, 0.0)
                  + jnp.where(lane == 2, ppos0, 0.0)
                  + jnp.where(lane == 3, ppos1, 0.0))
    ni = jax.lax.broadcasted_iota(jnp.int32, (NT, 128), 0)
    nl = jax.lax.broadcasted_iota(jnp.int32, (NT, 128), 1)
    tstart = jax.lax.convert_element_type(ni * TR, F32)
    ple = jnp.where(nl < NEXP, jnp.where(pstarts <= tstart, 1.0, 0.0), 0.0)
    te = jnp.sum(ple, axis=-1, keepdims=True) - 1.0
    t_ref[...] = jnp.where(nl == 0, te, 0.0)


def _gate(x, gw_pad, gb_pad):
    return pl.pallas_call(
        _gate_body,
        grid=(1,),
        in_specs=[pl.BlockSpec((S, DIM), lambda i: (0, 0)),
                  pl.BlockSpec((NEXP, DIM), lambda i: (0, 0)),
                  pl.BlockSpec((1, NEXP), lambda i: (0, 0))],
        out_specs=(pl.BlockSpec((S, 128), lambda i: (0, 0)),
                   pl.BlockSpec((NT, 128), lambda i: (0, 0))),
        out_shape=(jax.ShapeDtypeStruct((S, 128), F32),
                   jax.ShapeDtypeStruct((NT, 128), F32)),
    )(x, gw_pad, gb_pad)


def _f1_body(te_ref, xg_ref, w1_ref, w3_ref, b1_ref, b3_ref, o_ref):
    x = xg_ref[...]
    h1 = _dtb(x, w1_ref[0, 0]) + b1_ref[0]
    h3 = _dtb(x, w3_ref[0, 0]) + b3_ref[0]
    o_ref[...] = h1 * jax.nn.sigmoid(h1) * h3


def _moe_f1_half(te, xg, ew1, ew3, eb1, eb3, c):
    grid_spec = pltpu.PrefetchScalarGridSpec(
        num_scalar_prefetch=1,
        grid=(NT,),
        in_specs=[
            pl.BlockSpec((TR, DIM), lambda t, te: (t, 0)),
            pl.BlockSpec((1, 1, FC, DIM), lambda t, te: (te[t], c, 0, 0)),
            pl.BlockSpec((1, 1, FC, DIM), lambda t, te: (te[t], c, 0, 0)),
            pl.BlockSpec((1, 1, FC), lambda t, te: (te[t], 0, 0)),
            pl.BlockSpec((1, 1, FC), lambda t, te: (te[t], 0, 0)),
        ],
        out_specs=pl.BlockSpec((TR, FC), lambda t, te: (t, 0)),
    )
    return pl.pallas_call(
        _f1_body, grid_spec=grid_spec,
        out_shape=jax.ShapeDtypeStruct((PMAX, FC), F32),
    )(te, xg,
      ew1.reshape(NEXP, 2, FC, DIM), ew3.reshape(NEXP, 2, FC, DIM),
      eb1[:, c * FC:(c + 1) * FC].reshape(NEXP, 1, FC),
      eb3[:, c * FC:(c + 1) * FC].reshape(NEXP, 1, FC))


def _f2_body(te_ref, a0_ref, a1_ref, w_ref, b_ref, o_ref):
    a = jnp.concatenate([a0_ref[...], a1_ref[...]], axis=-1)
    o_ref[...] = _dtb(a, w_ref[0]) + b_ref[0]


def _moe_f2(te, act0, act1, ew2, eb2):
    grid_spec = pltpu.PrefetchScalarGridSpec(
        num_scalar_prefetch=1,
        grid=(NT,),
        in_specs=[
            pl.BlockSpec((TR, FC), lambda t, te: (t, 0)),
            pl.BlockSpec((TR, FC), lambda t, te: (t, 0)),
            pl.BlockSpec((1, DIM, INTER), lambda t, te: (te[t], 0, 0)),
            pl.BlockSpec((1, 1, DIM), lambda t, te: (te[t], 0, 0)),
        ],
        out_specs=pl.BlockSpec((TR, DIM), lambda t, te: (t, 0)),
    )
    return pl.pallas_call(
        _f2_body, grid_spec=grid_spec,
        out_shape=jax.ShapeDtypeStruct((PMAX, DIM), F32),
    )(te, act0, act1, ew2, eb2.reshape(NEXP, 1, DIM))


def _dmm_body(x_ref, w1_ref, b1_ref, w3_ref, b3_ref, o_ref):
    x = x_ref[...]
    h1 = _dtb(x, w1_ref[...]) + b1_ref[...]
    h3 = _dtb(x, w3_ref[...]) + b3_ref[...]
    o_ref[...] = h1 * jax.nn.sigmoid(h1) * h3


def _sh_act(x, sw1, sb1, sw3, sb3):
    return pl.pallas_call(
        _dmm_body,
        grid=(NSH, S // BM),
        in_specs=[pl.BlockSpec((BM, DIM), lambda n, m: (m, 0)),
                  pl.BlockSpec((BSH, DIM), lambda n, m: (n, 0)),
                  pl.BlockSpec((1, BSH), lambda n, m: (0, n)),
                  pl.BlockSpec((BSH, DIM), lambda n, m: (n, 0)),
                  pl.BlockSpec((1, BSH), lambda n, m: (0, n))],
        out_specs=pl.BlockSpec((BM, BSH), lambda n, m: (m, n)),
        out_shape=jax.ShapeDtypeStruct((S, SH_INTER), F32),
    )(x, sw1, sb1.reshape(1, SH_INTER), sw3, sb3.reshape(1, SH_INTER))


def _zf_body(a_ref, w_ref, b_ref, r_ref, g_ref, y0_ref, y1_ref, o_ref):
    o_ref[...] = (_dtb(a_ref[...], w_ref[...]) + b_ref[...] + r_ref[...]
                  + g_ref[:, 0:1] * y0_ref[...] + g_ref[:, 1:2] * y1_ref[...])


def _z_final(a, sw2, sb2, xa, gt, yg0, yg1):
    return pl.pallas_call(
        _zf_body,
        grid=(NZ, S // BM),
        in_specs=[pl.BlockSpec((BM, SH_INTER), lambda n, m: (m, 0)),
                  pl.BlockSpec((BZ, SH_INTER), lambda n, m: (n, 0)),
                  pl.BlockSpec((1, BZ), lambda n, m: (0, n)),
                  pl.BlockSpec((BM, BZ), lambda n, m: (m, n)),
                  pl.BlockSpec((BM, 128), lambda n, m: (m, 0)),
                  pl.BlockSpec((BM, BZ), lambda n, m: (m, n)),
                  pl.BlockSpec((BM, BZ), lambda n, m: (m, n))],
        out_specs=pl.BlockSpec((BM, BZ), lambda n, m: (m, n)),
        out_shape=jax.ShapeDtypeStruct((S, DIM), F32),
    )(a, sw2, sb2.reshape(1, DIM), xa, gt, yg0, yg1)


def kernel(x, start_pos, mask, wq_w, wq_b, wk_w, wk_b, wv_w, wv_b, wo_w, wo_b,
           an_w, fn_w, g_w, g_b, ew1, eb1, ew2, eb2, ew3, eb3,
           sw1, sb1, sw2, sb2, sw3, sb3):
    x2 = x[0]
    h, q = _lnmm(x2, an_w, wq_w, wq_b)
    k = _mm(h, wk_w, wk_b)
    v = _mm(h, wv_w, wv_b)
    ao = _attn(q, k, v, mask)
    xa, h2 = _mm_res(ao, wo_w, wo_b, x2, fn_w)

    gt, te_out = _gate(h2, g_w, g_b.reshape(1, NEXP))
    pos0 = gt[:, 2].astype(jnp.int32)
    pos1 = gt[:, 3].astype(jnp.int32)
    tile_e = te_out[:, 0].astype(jnp.int32)
    toks = jnp.arange(S, dtype=jnp.int32)
    tok_pad = (jnp.zeros((PMAX,), jnp.int32)
               .at[jnp.concatenate([pos0, pos1])]
               .set(jnp.concatenate([toks, toks])))

    xg = jnp.take(h2, tok_pad, axis=0)
    act0 = _moe_f1_half(tile_e, xg, ew1, ew3, eb1, eb3, 0)
    act1 = _moe_f1_half(tile_e, xg, ew1, ew3, eb1, eb3, 1)
    ye = _moe_f2(tile_e, act0, act1, ew2, eb2)
    yg0 = jnp.take(ye, pos0, axis=0)
    yg1 = jnp.take(ye, pos1, axis=0)

    sh = _sh_act(h2, sw1, sb1, sw3, sb3)
    out = _z_final(sh, sw2, sb2, xa, gt, yg0, yg1)
    return out[None]
